# baseline (speedup 1.0000x reference)
import numpy as np

import jax
import jax.numpy as jnp
from jax import lax
from jax.experimental import pallas as pl
from jax.experimental.pallas import tpu as pltpu
from jax.experimental.pallas import tpu_sc as plsc

B, H, W = 32, 512, 512
N = B * H * W
ROWS, COLS = 16384, 512

NT = 16
PER_TILE = N // NT
TROWS = ROWS // NT
CH = 8192
CROWS = CH // COLS
NCH = PER_TILE // CH
HB = 2048
PAD = 17
BIG = np.int32(2**30)


def _sc_select(pred_hbm, targ_hbm, maskf_hbm, out_hbm, res_hbm,
               bufp, buft, bufm, bufo, hist, mred, mgbuf, macc, outv,
               sh_hist, semi0, semi1, semo0, semo1):
    tid = lax.axis_index("s")
    base = tid * PER_TILE
    rbase = tid * TROWS
    iota16 = lax.iota(jnp.int32, 16)
    ones_i = jnp.ones((16,), jnp.int32)
    zeros_i = jnp.zeros((16,), jnp.int32)
    semi = (semi0, semi1)
    semo = (semo0, semo1)
    inbufs = (bufp, buft, bufm)

    @plsc.parallel_loop(0, HB * PAD // 16, unroll=8)
    def _(i):
        hist[pl.ds(i * 16, 16)] = zeros_i

    def start3(c, par):
        r0 = rbase + c * CROWS
        for hb, vb in ((pred_hbm, bufp), (targ_hbm, buft), (maskf_hbm, bufm)):
            pltpu.async_copy(hb.at[pl.ds(r0, CROWS)],
                             vb.at[pl.ds(par * CROWS, CROWS)], semi[par])

    def wait3(c, par):
        r0 = rbase + c * CROWS
        for hb, vb in ((pred_hbm, bufp), (targ_hbm, buft), (maskf_hbm, bufm)):
            pltpu.make_async_copy(hb.at[pl.ds(r0, CROWS)],
                                  vb.at[pl.ds(par * CROWS, CROWS)],
                                  semi[par]).wait()

    def p1_compute(c, par, cnt):
        @plsc.parallel_loop(0, CH // 16, 4, carry=cnt)
        def out(i, cv):
            cs = []
            for u in range(4):
                iu = i + u
                rr = par * CROWS + lax.shift_right_logical(iu, 5)
                cc = lax.shift_left(jnp.bitwise_and(iu, jnp.int32(31)), 4)
                p = bufp[rr, pl.ds(cc, 16)]
                t = buft[rr, pl.ds(cc, 16)]
                m = bufm[rr, pl.ds(cc, 16)]
                valid = m > jnp.float32(0.5)
                r = jnp.where(valid, jnp.abs(p - t), jnp.float32(1e30))
                bufo[par, pl.ds(iu * 16, 16)] = r
                bits = plsc.bitcast(r, jnp.int32)
                idx = lax.shift_right_logical(bits, 21) * PAD + iota16
                plsc.addupdate_scatter(hist, [idx], ones_i)
                cs.append(jnp.where(valid, 1, 0))
            return cv + ((cs[0] + cs[1]) + (cs[2] + cs[3]))
        return out

    start3(0, 0)
    cnt0 = jnp.zeros((16,), jnp.int32)

    def p1_body(j, cnt):
        c0 = 2 * j
        wait3(c0, 0)
        start3(c0 + 1, 1)
        cnt = p1_compute(c0, 0, cnt)

        @pl.when(j >= 1)
        def _():
            pltpu.make_async_copy(bufo.at[0], res_hbm.at[pl.ds(0, CH)],
                                  semo[0]).wait()
        pltpu.async_copy(bufo.at[0], res_hbm.at[pl.ds(base + c0 * CH, CH)],
                         semo[0])

        wait3(c0 + 1, 1)

        @pl.when(j < NCH // 2 - 1)
        def _():
            start3(c0 + 2, 0)
        cnt = p1_compute(c0 + 1, 1, cnt)

        @pl.when(j >= 1)
        def _():
            pltpu.make_async_copy(bufo.at[1], res_hbm.at[pl.ds(0, CH)],
                                  semo[1]).wait()
        pltpu.async_copy(bufo.at[1], res_hbm.at[pl.ds(base + (c0 + 1) * CH, CH)],
                         semo[1])
        return cnt

    cnt = lax.fori_loop(0, NCH // 2, p1_body, cnt0)
    pltpu.make_async_copy(bufo.at[0], res_hbm.at[pl.ds(0, CH)], semo[0]).wait()
    pltpu.make_async_copy(bufo.at[1], res_hbm.at[pl.ds(0, CH)], semo[1]).wait()

    def lane_reduce():
        @plsc.parallel_loop(0, HB // 16, unroll=2)
        def _(i):
            rows = (i * 16 + iota16) * PAD
            acc = plsc.load_gather(hist, [rows])
            for l in range(1, 16):
                acc = acc + plsc.load_gather(hist, [rows + l])
            mred[pl.ds(i * 16, 16)] = acc

    def merge():
        pltpu.sync_copy(mred, sh_hist.at[tid])
        plsc.subcore_barrier()
        for half in range(2):
            pltpu.sync_copy(sh_hist.at[pl.ds(half * 8, 8)], mgbuf)

            @plsc.parallel_loop(0, HB // 16, unroll=2)
            def _(i):
                acc = mgbuf[0, pl.ds(i * 16, 16)]
                for u in range(1, 8):
                    acc = acc + mgbuf[u, pl.ds(i * 16, 16)]
                if half == 0:
                    macc[pl.ds(i * 16, 16)] = acc
                else:
                    plsc.addupdate(macc.at[pl.ds(i * 16, 16)], acc)
        plsc.subcore_barrier()

    lane_reduce()
    cnt_s = jnp.sum(cnt)
    mred[pl.ds(1024, 16)] = jnp.where(iota16 == 0, cnt_s, 0)
    merge()

    nvv = macc[pl.ds(1024, 16)]
    num_valid = nvv[0]
    nv4 = num_valid * 4
    q0 = (num_valid.astype(jnp.float32) * jnp.float32(0.8)).astype(jnp.int32)
    keep = jnp.int32(0)
    for d in range(-2, 3):
        cand = q0 + d
        ok = jnp.logical_and(cand >= 0, cand * 5 <= nv4)
        keep = jnp.where(ok, jnp.maximum(keep, cand), keep)

    def choose(k_res):
        def b1(i, carry):
            total, best = carry
            v = macc[pl.ds(i * 16, 16)]
            cum = plsc.cumsum(v) + total
            lanes = iota16 + i * 16
            cand = jnp.min(jnp.where(cum >= k_res, lanes, BIG))
            return jnp.max(cum), jnp.minimum(best, cand)

        _, best = lax.fori_loop(0, HB // 16, b1, (jnp.int32(0), BIG))

        def b2(i, acc):
            v = macc[pl.ds(i * 16, 16)]
            lanes = iota16 + i * 16
            return acc + jnp.sum(jnp.where(lanes < best, v, 0))

        below = lax.fori_loop(0, HB // 16, b2, jnp.int32(0))
        return best, k_res - below

    b1sel, k1 = choose(keep)

    def start_r(c, par):
        pltpu.async_copy(res_hbm.at[pl.ds(base + c * CH, CH)],
                         bufo.at[par], semi[par])

    def wait_r(c, par):
        pltpu.make_async_copy(res_hbm.at[pl.ds(base + c * CH, CH)],
                              bufo.at[par], semi[par]).wait()

    def scan(process, carry_init):
        start_r(0, 0)

        def body(j, carry):
            wait_r(2 * j, 0)
            start_r(2 * j + 1, 1)
            carry = process(0, carry)
            wait_r(2 * j + 1, 1)

            @pl.when(j < NCH // 2 - 1)
            def _():
                start_r(2 * j + 2, 0)

            return process(1, carry)

        return lax.fori_loop(0, NCH // 2, body, carry_init)

    def zero_hist():
        @plsc.parallel_loop(0, HB * PAD // 16, unroll=8)
        def _(i):
            hist[pl.ds(i * 16, 16)] = zeros_i

    zero_hist()

    def process2(b, carry):
        @plsc.parallel_loop(0, CH // 16, 4, carry=carry)
        def out(i, sa):
            parts = []
            for u in range(4):
                v = bufo[b, pl.ds((i + u) * 16, 16)]
                bits = plsc.bitcast(v, jnp.int32)
                top = lax.shift_right_logical(bits, 21)
                bk = jnp.bitwise_and(lax.shift_right_logical(bits, 10),
                                     jnp.int32(0x7FF))
                plsc.addupdate_scatter(hist, [bk * PAD + iota16], ones_i,
                                       mask=top == b1sel)
                parts.append(jnp.where(top < b1sel, v, jnp.float32(0.0)))
            return sa + ((parts[0] + parts[1]) + (parts[2] + parts[3]))
        return out

    sa = scan(process2, jnp.zeros((16,), jnp.float32))
    lane_reduce()
    merge()
    b2sel, k2 = choose(k1)
    p2 = jnp.bitwise_or(lax.shift_left(b1sel, 11), b2sel)

    zero_hist()

    @plsc.parallel_loop(0, 1024, unroll=8)
    def _(i):
        bufp[lax.shift_right_logical(i, 5),
             pl.ds(lax.shift_left(jnp.bitwise_and(i, jnp.int32(31)), 4), 16)] \
            = jnp.zeros((16,), jnp.float32)

    def process3(b, carry):
        @plsc.parallel_loop(0, CH // 16, 4, carry=carry)
        def out(i, sb):
            parts = []
            for u in range(4):
                v = bufo[b, pl.ds((i + u) * 16, 16)]
                bits = plsc.bitcast(v, jnp.int32)
                pre = lax.shift_right_logical(bits, 10)
                match = pre == p2
                bk = jnp.bitwise_and(bits, jnp.int32(0x3FF))
                plsc.addupdate_scatter(hist, [bk * PAD + iota16], ones_i,
                                       mask=match)
                s = lax.shift_left(bk, 4) + iota16
                plsc.addupdate_scatter(
                    bufp,
                    [lax.shift_right_logical(s, 9),
                     jnp.bitwise_and(s, jnp.int32(511))],
                    v, mask=match)
                below_b = jnp.logical_and(
                    pre < p2, lax.shift_right_logical(pre, 11) == b1sel)
                parts.append(jnp.where(below_b, v, jnp.float32(0.0)))
            return sb + ((parts[0] + parts[1]) + (parts[2] + parts[3]))
        return out

    sb = scan(process3, jnp.zeros((16,), jnp.float32))
    lane_reduce()
    merge()
    b3sel, k3 = choose(k2)
    vbits = jnp.bitwise_or(lax.shift_left(p2, 10), b3sel)
    vcv = plsc.bitcast(jnp.full((16,), vbits, jnp.int32), jnp.float32)

    def sc_body(i, acc):
        w = bufp[lax.shift_right_logical(i, 5),
                 pl.ds(lax.shift_left(jnp.bitwise_and(i, jnp.int32(31)), 4), 16)]
        m = jnp.full((16,), i, jnp.int32) < b3sel
        return acc + jnp.where(m, w, jnp.float32(0.0))

    s_c = lax.fori_loop(0, 1024, sc_body, jnp.zeros((16,), jnp.float32))

    sl = jnp.sum(sa) + jnp.sum(sb) + jnp.sum(s_c)
    cnt_less = (keep - k3).astype(jnp.float32)
    pv = jnp.where(iota16 == 0, sl, jnp.float32(0.0))
    mred[pl.ds(0, 16)] = plsc.bitcast(pv, jnp.int32)
    pltpu.sync_copy(mred, sh_hist.at[tid])
    plsc.subcore_barrier()

    @pl.when(tid == 0)
    def _():
        pltpu.sync_copy(sh_hist.at[pl.ds(0, 8)], mgbuf)
        tot = jnp.zeros((16,), jnp.float32)
        for u in range(8):
            tot = tot + plsc.bitcast(mgbuf[u, pl.ds(0, 16)], jnp.float32)
        pltpu.sync_copy(sh_hist.at[pl.ds(8, 8)], mgbuf)
        for u in range(8):
            tot = tot + plsc.bitcast(mgbuf[u, pl.ds(0, 16)], jnp.float32)
        sum_less = jnp.full((16,), tot[0], jnp.float32)
        clv = jnp.full((16,), cnt_less, jnp.float32)
        kf = jnp.full((16,), keep.astype(jnp.float32), jnp.float32)
        trimmed = sum_less + (kf - clv) * vcv
        nvf = jnp.full((16,), num_valid.astype(jnp.float32), jnp.float32)
        divisor = jnp.maximum(nvf, jnp.float32(1.0))
        result = jnp.where(keep > 0, trimmed / divisor,
                           jnp.zeros((16,), jnp.float32))
        outv[...] = result
        pltpu.sync_copy(outv, out_hbm)


@jax.jit
def kernel(prediction, target, mask):
    p = prediction.reshape(ROWS, COLS)
    t = target.reshape(ROWS, COLS)
    mf = mask.astype(jnp.float32).reshape(ROWS, COLS)

    sc = pl.kernel(
        _sc_select,
        out_type=(jax.ShapeDtypeStruct((16,), jnp.float32),
                  jax.ShapeDtypeStruct((N,), jnp.float32)),
        mesh=plsc.VectorSubcoreMesh(core_axis_name="c", subcore_axis_name="s",
                                    num_cores=1),
        compiler_params=pltpu.CompilerParams(needs_layout_passes=False),
        scratch_types=[
            pltpu.VMEM((2 * CROWS, COLS), jnp.float32),
            pltpu.VMEM((2 * CROWS, COLS), jnp.float32),
            pltpu.VMEM((2 * CROWS, COLS), jnp.float32),
            pltpu.VMEM((2, CH), jnp.float32),
            pltpu.VMEM((HB * PAD,), jnp.int32),
            pltpu.VMEM((HB,), jnp.int32),
            pltpu.VMEM((8, HB), jnp.int32),
            pltpu.VMEM((HB,), jnp.int32),
            pltpu.VMEM((16,), jnp.float32),
            pltpu.VMEM_SHARED((NT, HB), jnp.int32),
            pltpu.SemaphoreType.DMA,
            pltpu.SemaphoreType.DMA,
            pltpu.SemaphoreType.DMA,
            pltpu.SemaphoreType.DMA,
        ],
    )
    out16, _ = sc(p, t, mf)
    return out16[0]


# speedup vs baseline: 1.1703x; 1.1703x over previous
import numpy as np

import jax
import jax.numpy as jnp
from jax import lax
from jax.experimental import pallas as pl
from jax.experimental.pallas import tpu as pltpu
from jax.experimental.pallas import tpu_sc as plsc

B, H, W = 32, 512, 512
N = B * H * W
ROWS, COLS = 16384, 512

NT = 16
PER_TILE = N // NT
TROWS = ROWS // NT
CH = 8192
CROWS = CH // COLS
NCH = PER_TILE // CH
HB = 2048
PAD = 17
BIG = np.int32(2**30)


def _sc_select(pred_hbm, targ_hbm, maskf_hbm, out_hbm, res_hbm,
               bufp, buft, bufm, bufo, hist, mred, mgbuf, macc, outv,
               sh_hist, semi0, semi1, semo0, semo1):
    tid = lax.axis_index("s")
    base = tid * PER_TILE
    rbase = tid * TROWS
    iota16 = lax.iota(jnp.int32, 16)
    ones_i = jnp.ones((16,), jnp.int32)
    zeros_i = jnp.zeros((16,), jnp.int32)
    semi = (semi0, semi1)
    semo = (semo0, semo1)
    inbufs = (bufp, buft, bufm)

    @plsc.parallel_loop(0, HB * PAD // 16, unroll=8)
    def _(i):
        hist[pl.ds(i * 16, 16)] = zeros_i

    def start3(c, par):
        r0 = rbase + c * CROWS
        for hb, vb in ((pred_hbm, bufp), (targ_hbm, buft), (maskf_hbm, bufm)):
            pltpu.async_copy(hb.at[pl.ds(r0, CROWS)],
                             vb.at[pl.ds(par * CROWS, CROWS)], semi[par])

    def wait3(c, par):
        r0 = rbase + c * CROWS
        for hb, vb in ((pred_hbm, bufp), (targ_hbm, buft), (maskf_hbm, bufm)):
            pltpu.make_async_copy(hb.at[pl.ds(r0, CROWS)],
                                  vb.at[pl.ds(par * CROWS, CROWS)],
                                  semi[par]).wait()

    def p1_compute(c, par, cnt):
        @plsc.parallel_loop(0, CH // 16, 4, carry=cnt)
        def out(i, cv):
            cs = []
            for u in range(4):
                iu = i + u
                rr = par * CROWS + lax.shift_right_logical(iu, 5)
                cc = lax.shift_left(jnp.bitwise_and(iu, jnp.int32(31)), 4)
                p = bufp[rr, pl.ds(cc, 16)]
                t = buft[rr, pl.ds(cc, 16)]
                m = bufm[rr, pl.ds(cc, 16)]
                valid = m > jnp.float32(0.5)
                r = jnp.where(valid, jnp.abs(p - t), jnp.float32(1e30))
                bufo[par, pl.ds(iu * 16, 16)] = r
                bits = plsc.bitcast(r, jnp.int32)
                idx = lax.shift_right_logical(bits, 21) * PAD + iota16
                plsc.addupdate_scatter(hist, [idx], ones_i)
                cs.append(jnp.where(valid, 1, 0))
            return cv + ((cs[0] + cs[1]) + (cs[2] + cs[3]))
        return out

    start3(0, 0)
    cnt0 = jnp.zeros((16,), jnp.int32)

    def p1_body(j, cnt):
        c0 = 2 * j
        wait3(c0, 0)
        start3(c0 + 1, 1)
        cnt = p1_compute(c0, 0, cnt)

        @pl.when(j >= 1)
        def _():
            pltpu.make_async_copy(bufo.at[0], res_hbm.at[pl.ds(0, CH)],
                                  semo[0]).wait()
        pltpu.async_copy(bufo.at[0], res_hbm.at[pl.ds(base + c0 * CH, CH)],
                         semo[0])

        wait3(c0 + 1, 1)

        @pl.when(j < NCH // 2 - 1)
        def _():
            start3(c0 + 2, 0)
        cnt = p1_compute(c0 + 1, 1, cnt)

        @pl.when(j >= 1)
        def _():
            pltpu.make_async_copy(bufo.at[1], res_hbm.at[pl.ds(0, CH)],
                                  semo[1]).wait()
        pltpu.async_copy(bufo.at[1], res_hbm.at[pl.ds(base + (c0 + 1) * CH, CH)],
                         semo[1])
        return cnt

    cnt = lax.fori_loop(0, NCH // 2, p1_body, cnt0)
    pltpu.make_async_copy(bufo.at[0], res_hbm.at[pl.ds(0, CH)], semo[0]).wait()
    pltpu.make_async_copy(bufo.at[1], res_hbm.at[pl.ds(0, CH)], semo[1]).wait()

    def lane_reduce():
        @plsc.parallel_loop(0, HB // 16, unroll=2)
        def _(i):
            rows = (i * 16 + iota16) * PAD
            acc = plsc.load_gather(hist, [rows])
            for l in range(1, 16):
                acc = acc + plsc.load_gather(hist, [rows + l])
            mred[pl.ds(i * 16, 16)] = acc

    def merge():
        pltpu.sync_copy(mred, sh_hist.at[tid])
        plsc.subcore_barrier()
        for half in range(2):
            pltpu.sync_copy(sh_hist.at[pl.ds(half * 8, 8)], mgbuf)

            @plsc.parallel_loop(0, HB // 16, unroll=2)
            def _(i):
                acc = mgbuf[0, pl.ds(i * 16, 16)]
                for u in range(1, 8):
                    acc = acc + mgbuf[u, pl.ds(i * 16, 16)]
                if half == 0:
                    macc[pl.ds(i * 16, 16)] = acc
                else:
                    plsc.addupdate(macc.at[pl.ds(i * 16, 16)], acc)
        plsc.subcore_barrier()

    lane_reduce()
    cnt_s = jnp.sum(cnt)
    mred[pl.ds(1024, 16)] = jnp.where(iota16 == 0, cnt_s, 0)
    merge()

    nvv = macc[pl.ds(1024, 16)]
    num_valid = nvv[0]
    nv4 = num_valid * 4
    q0 = (num_valid.astype(jnp.float32) * jnp.float32(0.8)).astype(jnp.int32)
    keep = jnp.int32(0)
    for d in range(-2, 3):
        cand = q0 + d
        ok = jnp.logical_and(cand >= 0, cand * 5 <= nv4)
        keep = jnp.where(ok, jnp.maximum(keep, cand), keep)

    def choose(k_res):
        def b1(i, carry):
            total, best = carry
            v = macc[pl.ds(i * 16, 16)]
            cum = plsc.cumsum(v) + total
            lanes = iota16 + i * 16
            cand = jnp.min(jnp.where(cum >= k_res, lanes, BIG))
            return jnp.max(cum), jnp.minimum(best, cand)

        _, best = lax.fori_loop(0, HB // 16, b1, (jnp.int32(0), BIG))

        def b2(i, acc):
            v = macc[pl.ds(i * 16, 16)]
            lanes = iota16 + i * 16
            return acc + jnp.sum(jnp.where(lanes < best, v, 0))

        below = lax.fori_loop(0, HB // 16, b2, jnp.int32(0))
        return best, k_res - below

    b1sel, k1 = choose(keep)

    def start_r(c, par):
        pltpu.async_copy(res_hbm.at[pl.ds(base + c * CH, CH)],
                         bufo.at[par], semi[par])

    def wait_r(c, par):
        pltpu.make_async_copy(res_hbm.at[pl.ds(base + c * CH, CH)],
                              bufo.at[par], semi[par]).wait()

    def scan(process, carry_init):
        start_r(0, 0)

        def body(j, carry):
            wait_r(2 * j, 0)
            start_r(2 * j + 1, 1)
            carry = process(0, carry)
            wait_r(2 * j + 1, 1)

            @pl.when(j < NCH // 2 - 1)
            def _():
                start_r(2 * j + 2, 0)

            return process(1, carry)

        return lax.fori_loop(0, NCH // 2, body, carry_init)

    def hist_pass(bucket_fn):
        @plsc.parallel_loop(0, HB * PAD // 16, unroll=8)
        def _(i):
            hist[pl.ds(i * 16, 16)] = zeros_i

        def process(b, carry):
            @plsc.parallel_loop(0, CH // 16, unroll=8)
            def _(i):
                v = bufo[b, pl.ds(i * 16, 16)]
                bits = plsc.bitcast(v, jnp.int32)
                bk, match = bucket_fn(bits)
                idx = bk * PAD + iota16
                plsc.addupdate_scatter(hist, [idx], ones_i, mask=match)
            return carry

        scan(process, 0)
        lane_reduce()
        merge()

    hist_pass(lambda bits: (
        jnp.bitwise_and(lax.shift_right_logical(bits, 10), jnp.int32(0x7FF)),
        lax.shift_right_logical(bits, 21) == b1sel))
    b2sel, k2 = choose(k1)
    p2 = jnp.bitwise_or(lax.shift_left(b1sel, 11), b2sel)

    hist_pass(lambda bits: (
        jnp.bitwise_and(bits, jnp.int32(0x3FF)),
        lax.shift_right_logical(bits, 10) == p2))
    b3sel, _ = choose(k2)
    vbits = jnp.bitwise_or(lax.shift_left(p2, 10), b3sel)
    vcv = plsc.bitcast(jnp.full((16,), vbits, jnp.int32), jnp.float32)

    def process4(b, carry):
        @plsc.parallel_loop(0, CH // 16, 8, carry=carry)
        def out(i, c):
            sumv, cntv = c
            sv, cv = [], []
            for u in range(8):
                v = bufo[b, pl.ds((i + u) * 16, 16)]
                m = v < vcv
                sv.append(jnp.where(m, v, jnp.float32(0.0)))
                cv.append(jnp.where(m, 1, 0))
            while len(sv) > 1:
                sv = [a + b2 for a, b2 in zip(sv[::2], sv[1::2])]
                cv = [a + b2 for a, b2 in zip(cv[::2], cv[1::2])]
            return sumv + sv[0], cntv + cv[0]
        return out

    sumv, cntv = scan(process4,
                      (jnp.zeros((16,), jnp.float32), jnp.zeros((16,), jnp.int32)))

    sl = jnp.sum(sumv)
    cl = jnp.sum(cntv).astype(jnp.float32)
    pv = (jnp.where(iota16 == 0, sl, jnp.float32(0.0))
          + jnp.where(iota16 == 1, cl, jnp.float32(0.0)))
    mred[pl.ds(0, 16)] = plsc.bitcast(pv, jnp.int32)
    pltpu.sync_copy(mred, sh_hist.at[tid])
    plsc.subcore_barrier()

    @pl.when(tid == 0)
    def _():
        pltpu.sync_copy(sh_hist.at[pl.ds(0, 8)], mgbuf)
        tot = jnp.zeros((16,), jnp.float32)
        for u in range(8):
            tot = tot + plsc.bitcast(mgbuf[u, pl.ds(0, 16)], jnp.float32)
        pltpu.sync_copy(sh_hist.at[pl.ds(8, 8)], mgbuf)
        for u in range(8):
            tot = tot + plsc.bitcast(mgbuf[u, pl.ds(0, 16)], jnp.float32)
        sum_less = jnp.full((16,), tot[0], jnp.float32)
        cnt_less = jnp.full((16,), tot[1], jnp.float32)
        kf = jnp.full((16,), keep.astype(jnp.float32), jnp.float32)
        trimmed = sum_less + (kf - cnt_less) * vcv
        nvf = jnp.full((16,), num_valid.astype(jnp.float32), jnp.float32)
        divisor = jnp.maximum(nvf, jnp.float32(1.0))
        result = jnp.where(keep > 0, trimmed / divisor,
                           jnp.zeros((16,), jnp.float32))
        outv[...] = result
        pltpu.sync_copy(outv, out_hbm)


@jax.jit
def kernel(prediction, target, mask):
    p = prediction.reshape(ROWS, COLS)
    t = target.reshape(ROWS, COLS)
    mf = mask.astype(jnp.float32).reshape(ROWS, COLS)

    sc = pl.kernel(
        _sc_select,
        out_type=(jax.ShapeDtypeStruct((16,), jnp.float32),
                  jax.ShapeDtypeStruct((N,), jnp.float32)),
        mesh=plsc.VectorSubcoreMesh(core_axis_name="c", subcore_axis_name="s",
                                    num_cores=1),
        compiler_params=pltpu.CompilerParams(needs_layout_passes=False),
        scratch_types=[
            pltpu.VMEM((2 * CROWS, COLS), jnp.float32),
            pltpu.VMEM((2 * CROWS, COLS), jnp.float32),
            pltpu.VMEM((2 * CROWS, COLS), jnp.float32),
            pltpu.VMEM((2, CH), jnp.float32),
            pltpu.VMEM((HB * PAD,), jnp.int32),
            pltpu.VMEM((HB,), jnp.int32),
            pltpu.VMEM((8, HB), jnp.int32),
            pltpu.VMEM((HB,), jnp.int32),
            pltpu.VMEM((16,), jnp.float32),
            pltpu.VMEM_SHARED((NT, HB), jnp.int32),
            pltpu.SemaphoreType.DMA,
            pltpu.SemaphoreType.DMA,
            pltpu.SemaphoreType.DMA,
            pltpu.SemaphoreType.DMA,
        ],
    )
    out16, _ = sc(p, t, mf)
    return out16[0]
